# Initial kernel scaffold; baseline (speedup 1.0000x reference)
import jax
import jax.numpy as jnp
from jax.experimental import pallas as pl


def kernel(pos, node_size_x, node_size_y, flop_indices):
    raise NotImplementedError("write your pallas kernel here")


# speedup vs baseline: 18.6965x; 18.6965x over previous
import functools

import jax
import jax.numpy as jnp
from jax import lax
from jax.experimental import pallas as pl
from jax.experimental.pallas import tpu as pltpu
from jax.experimental.pallas import tpu_sc as plsc

N = 1_000_000
HALF = N // 2
NSC = 2
NT = 16
PER_TILE = 25_088
PAD_TOT = NT * PER_TILE
SP = 500_224
ZR = SP // NT
ZHALF = ZR // 2
WB_LAST = HALF - ZR


def _sc_body(nsx_hbm, nsy_hbm, idx_hbm, out_hbm,
             idx_v, gx_v, gy_v, zbuf, spmem, sem1, sem2):
    c = lax.axis_index("c")
    s = lax.axis_index("s")
    base = c * HALF
    dummy = HALF + s * 8

    def zloop(i, _):
        zbuf[pl.ds(i * 16, 16)] = jnp.zeros((16,), jnp.float32)
        return 0
    lax.fori_loop(0, ZHALF // 16, zloop, 0)
    pltpu.sync_copy(zbuf, spmem.at[pl.ds(s * ZR, ZHALF)])
    pltpu.sync_copy(zbuf, spmem.at[pl.ds(s * ZR + ZHALF, ZHALF)])

    pltpu.sync_copy(idx_hbm.at[s], idx_v)
    cp1 = pltpu.async_copy(nsx_hbm.at[idx_v], gx_v, sem1)
    cp2 = pltpu.async_copy(nsy_hbm.at[idx_v], gy_v, sem2)
    cp1.wait()
    cp2.wait()

    def cloop(j, _):
        for k in range(8):
            sl = pl.ds(j * 128 + k * 16, 16)
            gx_v[sl] = gx_v[sl] * gy_v[sl]
            rel = idx_v[sl] - base
            ok = (rel >= 0) & (rel < HALF)
            idx_v[sl] = jnp.where(ok, rel, dummy)
        return 0
    lax.fori_loop(0, PER_TILE // 128, cloop, 0)

    plsc.subcore_barrier()
    pltpu.sync_copy(gx_v, spmem.at[idx_v])
    plsc.subcore_barrier()

    off = jnp.minimum(s * ZR, WB_LAST)
    for t in range(2):
        pltpu.sync_copy(spmem.at[pl.ds(off + t * ZHALF, ZHALF)], zbuf)
        pltpu.sync_copy(zbuf, out_hbm.at[pl.ds(base + off + t * ZHALF, ZHALF)])


def kernel(pos, node_size_x, node_size_y, flop_indices):
    del pos
    idxp = jnp.concatenate(
        [flop_indices, flop_indices[: PAD_TOT - flop_indices.shape[0]]]
    ).reshape(NT, PER_TILE)

    mesh = plsc.VectorSubcoreMesh(core_axis_name="c", subcore_axis_name="s")
    call = functools.partial(
        pl.kernel,
        out_type=jax.ShapeDtypeStruct((N,), jnp.float32),
        mesh=mesh,
        scratch_types=[
            pltpu.VMEM((PER_TILE,), jnp.int32),
            pltpu.VMEM((PER_TILE,), jnp.float32),
            pltpu.VMEM((PER_TILE,), jnp.float32),
            pltpu.VMEM((ZHALF,), jnp.float32),
            pltpu.VMEM_SHARED((SP,), jnp.float32),
            pltpu.SemaphoreType.DMA,
            pltpu.SemaphoreType.DMA,
        ],
    )(_sc_body)
    return call(node_size_x, node_size_y, idxp)


# speedup vs baseline: 19.3186x; 1.0333x over previous
import functools

import jax
import jax.numpy as jnp
from jax import lax
from jax.experimental import pallas as pl
from jax.experimental.pallas import tpu as pltpu
from jax.experimental.pallas import tpu_sc as plsc

N = 1_000_000
HALF = N // 2
NSC = 2
NT = 16
PER_TILE = 25_088
PAD_TOT = NT * PER_TILE
SP = 500_224
ZR = SP // NT
ZHALF = ZR // 2
WB_LAST = HALF - ZR


def _sc_body(nsx_hbm, nsy_hbm, idx_hbm, out_hbm,
             idx_v, gx_v, gy_v, zbuf, spmem, sem1, sem2):
    c = lax.axis_index("c")
    s = lax.axis_index("s")
    base = c * HALF
    dummy = HALF + s * 8

    cp0 = pltpu.async_copy(idx_hbm.at[s], idx_v, sem1)

    def zloop(i, _):
        zbuf[pl.ds(i * 16, 16)] = jnp.zeros((16,), jnp.float32)
        return 0
    lax.fori_loop(0, ZHALF // 16, zloop, 0)
    cp0.wait()

    cp1 = pltpu.async_copy(nsx_hbm.at[idx_v], gx_v, sem1)
    cp2 = pltpu.async_copy(nsy_hbm.at[idx_v], gy_v, sem2)
    pltpu.sync_copy(zbuf, spmem.at[pl.ds(s * ZR, ZHALF)])
    pltpu.sync_copy(zbuf, spmem.at[pl.ds(s * ZR + ZHALF, ZHALF)])
    cp1.wait()
    cp2.wait()

    def cloop(j, _):
        for k in range(8):
            sl = pl.ds(j * 128 + k * 16, 16)
            gx_v[sl] = gx_v[sl] * gy_v[sl]
            rel = idx_v[sl] - base
            ok = (rel >= 0) & (rel < HALF)
            idx_v[sl] = jnp.where(ok, rel, dummy)
        return 0
    lax.fori_loop(0, PER_TILE // 128, cloop, 0)

    plsc.subcore_barrier()
    pltpu.sync_copy(gx_v, spmem.at[idx_v])
    plsc.subcore_barrier()

    off = jnp.minimum(s * ZR, WB_LAST)
    for t in range(2):
        pltpu.sync_copy(spmem.at[pl.ds(off + t * ZHALF, ZHALF)], zbuf)
        pltpu.sync_copy(zbuf, out_hbm.at[pl.ds(base + off + t * ZHALF, ZHALF)])


def kernel(pos, node_size_x, node_size_y, flop_indices):
    del pos
    idxp = jnp.concatenate(
        [flop_indices, flop_indices[: PAD_TOT - flop_indices.shape[0]]]
    ).reshape(NT, PER_TILE)

    mesh = plsc.VectorSubcoreMesh(core_axis_name="c", subcore_axis_name="s")
    call = functools.partial(
        pl.kernel,
        out_type=jax.ShapeDtypeStruct((N,), jnp.float32),
        mesh=mesh,
        scratch_types=[
            pltpu.VMEM((PER_TILE,), jnp.int32),
            pltpu.VMEM((PER_TILE,), jnp.float32),
            pltpu.VMEM((PER_TILE,), jnp.float32),
            pltpu.VMEM((ZHALF,), jnp.float32),
            pltpu.VMEM_SHARED((SP,), jnp.float32),
            pltpu.SemaphoreType.DMA,
            pltpu.SemaphoreType.DMA,
        ],
    )(_sc_body)
    return call(node_size_x, node_size_y, idxp)


# speedup vs baseline: 31.0231x; 1.6059x over previous
import functools

import jax
import jax.numpy as jnp
from jax import lax
from jax.experimental import pallas as pl
from jax.experimental.pallas import tpu as pltpu
from jax.experimental.pallas import tpu_sc as plsc

N = 1_000_000
HALF = N // 2
NT = 16
PER_TILE = 25_088
PAD_TOT = NT * PER_TILE
SP = 500_224
ZR = SP // NT
ZHALF = ZR // 2
WB_LAST = HALF - ZR


def _sc_body(nsx_hbm, nsy_hbm, idx_hbm, out_hbm,
             idx_v, g_v, zbuf, sp_areas, sp_out, sem1, sem2, sem3):
    c = lax.axis_index("c")
    s = lax.axis_index("s")
    base = c * HALF
    dummy = HALF + s * 8

    cp0 = pltpu.async_copy(idx_hbm.at[s], idx_v, sem1)

    aoff = jnp.minimum(s * ZR, WB_LAST)
    for t in range(2):
        po = aoff + t * ZHALF
        cpx = pltpu.async_copy(nsx_hbm.at[pl.ds(base + po, ZHALF)],
                               g_v.at[pl.ds(0, ZHALF)], sem2)
        cpy = pltpu.async_copy(nsy_hbm.at[pl.ds(base + po, ZHALF)],
                               zbuf, sem3)
        cpx.wait()
        cpy.wait()

        def mloop(i, _):
            sl = pl.ds(i * 16, 16)
            g_v[sl] = g_v[sl] * zbuf[sl]
            return 0
        lax.fori_loop(0, ZHALF // 16, mloop, 0)
        pltpu.sync_copy(g_v.at[pl.ds(0, ZHALF)],
                        sp_areas.at[pl.ds(po, ZHALF)])

    def zloop(i, _):
        zbuf[pl.ds(i * 16, 16)] = jnp.zeros((16,), jnp.float32)
        return 0
    lax.fori_loop(0, ZHALF // 16, zloop, 0)
    pltpu.sync_copy(zbuf, sp_out.at[pl.ds(s * ZR, ZHALF)])
    pltpu.sync_copy(zbuf, sp_out.at[pl.ds(s * ZR + ZHALF, ZHALF)])

    cp0.wait()

    def cloop(j, _):
        for k in range(8):
            sl = pl.ds(j * 128 + k * 16, 16)
            rel = idx_v[sl] - base
            ok = (rel >= 0) & (rel < HALF)
            idx_v[sl] = jnp.where(ok, rel, dummy)
        return 0
    lax.fori_loop(0, PER_TILE // 128, cloop, 0)

    plsc.subcore_barrier()
    pltpu.async_copy(sp_areas.at[idx_v], g_v, sem2).wait()
    pltpu.sync_copy(g_v, sp_out.at[idx_v])
    plsc.subcore_barrier()

    off = jnp.minimum(s * ZR, WB_LAST)
    for t in range(2):
        pltpu.sync_copy(sp_out.at[pl.ds(off + t * ZHALF, ZHALF)], zbuf)
        pltpu.sync_copy(zbuf, out_hbm.at[pl.ds(base + off + t * ZHALF, ZHALF)])


def kernel(pos, node_size_x, node_size_y, flop_indices):
    del pos
    idxp = jnp.concatenate(
        [flop_indices, flop_indices[: PAD_TOT - flop_indices.shape[0]]]
    ).reshape(NT, PER_TILE)

    mesh = plsc.VectorSubcoreMesh(core_axis_name="c", subcore_axis_name="s")
    call = functools.partial(
        pl.kernel,
        out_type=jax.ShapeDtypeStruct((N,), jnp.float32),
        mesh=mesh,
        scratch_types=[
            pltpu.VMEM((PER_TILE,), jnp.int32),
            pltpu.VMEM((PER_TILE,), jnp.float32),
            pltpu.VMEM((ZHALF,), jnp.float32),
            pltpu.VMEM_SHARED((SP,), jnp.float32),
            pltpu.VMEM_SHARED((SP,), jnp.float32),
            pltpu.SemaphoreType.DMA,
            pltpu.SemaphoreType.DMA,
            pltpu.SemaphoreType.DMA,
        ],
    )(_sc_body)
    return call(node_size_x, node_size_y, idxp)
